# baseline (speedup 1.0000x reference)
import functools

import jax
import jax.numpy as jnp
from jax import lax
from jax.experimental import pallas as pl
from jax.experimental.pallas import tpu as pltpu
from jax.experimental.pallas import tpu_sc as plsc

_D = 32
_B = 16384
_UREG = 0.0025
_PREG = 0.0025
_NREG = 0.00025

_NC = 2
_NS = 16
_NW = _NC * _NS
_BPW = _B // _NW
_CH = 128
_NCH = _BPW // _CH
_NG = _BPW // 16


def _sc_gather_score(user_ids, pos_ids, neg_ids, user_table, item_table):
    mesh = plsc.VectorSubcoreMesh(core_axis_name="c", subcore_axis_name="s",
                                  num_cores=_NC, num_subcores=_NS)
    out_type = (
        jax.ShapeDtypeStruct((_B,), jnp.float32),
        jax.ShapeDtypeStruct((_B,), jnp.float32),
        jax.ShapeDtypeStruct((_NW, 16), jnp.float32),
    )
    scratch = [
        pltpu.VMEM((_NCH, _CH), jnp.int32),
        pltpu.VMEM((_NCH, _CH), jnp.int32),
        pltpu.VMEM((_NCH, _CH), jnp.int32),
        pltpu.VMEM((_BPW, _D), jnp.float32),
        pltpu.VMEM((_BPW, _D), jnp.float32),
        pltpu.VMEM((_BPW, _D), jnp.float32),
        pltpu.VMEM((_BPW,), jnp.float32),
        pltpu.VMEM((_BPW,), jnp.float32),
        pltpu.VMEM((16,), jnp.float32),
        pltpu.SemaphoreType.DMA,
    ]

    @functools.partial(pl.kernel, mesh=mesh, out_type=out_type,
                       scratch_types=scratch,
                       compiler_params=pltpu.CompilerParams(
                           needs_layout_passes=False,
                           use_tc_tiling_on_sc=False))
    def k(uids, pids, nids, utab, itab, posd_out, negd_out, reg_out,
          uidx, pidx, nidx, urows, prows, nrows, posd, negd, regv, sem):
        wid = lax.axis_index("s") * _NC + lax.axis_index("c")

        pltpu.sync_copy(uids.at[wid], uidx)
        pltpu.sync_copy(pids.at[wid], pidx)
        pltpu.sync_copy(nids.at[wid], nidx)

        copies = []
        for j in range(_NCH):
            sl = pl.ds(j * _CH, _CH)
            copies.append(pltpu.async_copy(utab.at[uidx.at[j]], urows.at[sl], sem))
            copies.append(pltpu.async_copy(itab.at[pidx.at[j]], prows.at[sl], sem))
            copies.append(pltpu.async_copy(itab.at[nidx.at[j]], nrows.at[sl], sem))
        for c in copies:
            c.wait()

        lanes = lax.iota(jnp.int32, 16)

        def group(g, reg_acc):
            rows = g * 16 + lanes
            pos_acc = jnp.zeros((16,), jnp.float32)
            neg_acc = jnp.zeros((16,), jnp.float32)
            for d in range(_D):
                col = jnp.full((16,), d, jnp.int32)
                u = plsc.load_gather(urows, [rows, col])
                p = plsc.load_gather(prows, [rows, col])
                q = plsc.load_gather(nrows, [rows, col])
                pos_acc = pos_acc + u * p
                neg_acc = neg_acc + u * q
                reg_acc = reg_acc + (_UREG * (u * u) + _PREG * (p * p)
                                     + _NREG * (q * q))
            posd[pl.ds(g * 16, 16)] = pos_acc
            negd[pl.ds(g * 16, 16)] = neg_acc
            return reg_acc

        reg_acc = lax.fori_loop(0, _NG, group, jnp.zeros((16,), jnp.float32))
        regv[...] = reg_acc

        pltpu.sync_copy(posd, posd_out.at[pl.ds(wid * _BPW, _BPW)])
        pltpu.sync_copy(negd, negd_out.at[pl.ds(wid * _BPW, _BPW)])
        pltpu.sync_copy(regv, reg_out.at[wid])

    return k(user_ids.reshape(_NW, _NCH, _CH),
             pos_ids.reshape(_NW, _NCH, _CH),
             neg_ids.reshape(_NW, _NCH, _CH),
             user_table, item_table)


def _finalize_body(pos_ref, neg_ref, reg_ref, out_ref):
    x = neg_ref[...] - pos_ref[...]
    sp = jnp.maximum(x, 0.0) + jnp.log(1.0 + jnp.exp(-jnp.abs(x)))
    out_ref[...] = (jnp.sum(sp, keepdims=True) / _B
                    + jnp.sum(reg_ref[...], keepdims=True))


def _finalize(posd, negd, regp):
    out = pl.pallas_call(
        _finalize_body,
        out_shape=jax.ShapeDtypeStruct((1, 1), jnp.float32),
    )(posd.reshape(128, 128), negd.reshape(128, 128), regp.reshape(4, 128))
    return out[0, 0]


def kernel(user_ids, pos_ids, neg_ids, user_table, item_table):
    posd, negd, regp = _sc_gather_score(user_ids, pos_ids, neg_ids,
                                        user_table, item_table)
    return _finalize(posd, negd, regp)
